# baseline (speedup 1.0000x reference)
import functools
import math

import jax
import jax.numpy as jnp
from jax import lax
from jax.experimental import pallas as pl
from jax.experimental.pallas import tpu as pltpu
from jax.experimental.pallas import tpu_sc as plsc

_NC = 2
_NS = 16
_NW = _NC * _NS
_LANES = 16

_IDXW = 64
_K = 1
_CHUNK = _K * _IDXW
_NBUF = 12


def kernel(x, lut):
    n_seq, seq_len = x.shape
    B = n_seq * seq_len
    V, D = lut.shape
    assert B % (_NW * _CHUNK) == 0 and D % _LANES == 0
    assert n_seq % _CHUNK == 0
    idx_per_w = B // _NW
    n_chunks = idx_per_w // _CHUNK
    chunks_per_row = n_seq // _CHUNK
    scale = math.sqrt(float(D))

    idx3d = x.T.reshape(B // (_K * _IDXW), _K, _IDXW).astype(jnp.int32)

    mesh = plsc.VectorSubcoreMesh(core_axis_name="c", subcore_axis_name="s")

    @functools.partial(
        pl.kernel,
        mesh=mesh,
        out_type=jax.ShapeDtypeStruct((seq_len, n_seq, D), jnp.float32),
        scratch_types=[
            pltpu.VMEM((n_chunks, _K, _IDXW), jnp.int32),
            pltpu.VMEM((_NBUF, _CHUNK, D), jnp.float32),
        ] + [pltpu.SemaphoreType.DMA] * (2 * _NBUF),
    )
    def emb(idx_hbm, table_hbm, out_hbm, idx_v, rows_v, *sems):
        gsem = sems[:_NBUF]
        wsem = sems[_NBUF:]
        wid = lax.axis_index("s") * _NC + lax.axis_index("c")
        chunk0 = wid * n_chunks

        pltpu.sync_copy(idx_hbm.at[pl.ds(chunk0, n_chunks)], idx_v)

        def start_chunk(g):
            s = g % _NBUF
            return [
                pltpu.async_copy(
                    table_hbm.at[idx_v.at[g, j]],
                    rows_v.at[s, pl.ds(j * _IDXW, _IDXW), :],
                    gsem[s])
                for j in range(_K)
            ]

        pending_g = {g: start_chunk(g) for g in range(min(_NBUF - 1, n_chunks))}
        pending_wb = [None] * _NBUF
        for g in range(n_chunks):
            s = g % _NBUF
            gn = g + _NBUF - 1
            if gn < n_chunks:
                sn = gn % _NBUF
                if pending_wb[sn] is not None:
                    pending_wb[sn].wait()
                    pending_wb[sn] = None
                pending_g[gn] = start_chunk(gn)
            for d in pending_g.pop(g):
                d.wait()

            def scale_row(i, carry, s=s):
                for j in range(D // _LANES):
                    sl = pl.ds(j * _LANES, _LANES)
                    rows_v[s, i, sl] = rows_v[s, i, sl] * scale
                return carry
            lax.fori_loop(0, _CHUNK, scale_row, 0)

            gchunk = chunk0 + g
            pending_wb[s] = pltpu.async_copy(
                rows_v.at[s],
                out_hbm.at[gchunk // chunks_per_row,
                           pl.ds((gchunk % chunks_per_row) * _CHUNK, _CHUNK),
                           :],
                wsem[s])
        for s in range(_NBUF):
            if pending_wb[s] is not None:
                pending_wb[s].wait()

    out_t = emb(idx3d, lut)
    return jnp.transpose(out_t, (1, 0, 2))


# speedup vs baseline: 1.0437x; 1.0437x over previous
import functools
import math

import jax
import jax.numpy as jnp
from jax import lax
from jax.experimental import pallas as pl
from jax.experimental.pallas import tpu as pltpu
from jax.experimental.pallas import tpu_sc as plsc

_NC = 2
_NS = 16
_NW = _NC * _NS
_LANES = 16

_IDXW = 128
_K = 1
_CHUNK = _K * _IDXW
_NBUF = 7


def kernel(x, lut):
    n_seq, seq_len = x.shape
    B = n_seq * seq_len
    V, D = lut.shape
    assert B % (_NW * _CHUNK) == 0 and D % _LANES == 0
    assert n_seq % _CHUNK == 0
    idx_per_w = B // _NW
    n_chunks = idx_per_w // _CHUNK
    chunks_per_row = n_seq // _CHUNK
    scale = math.sqrt(float(D))

    idx3d = x.T.reshape(B // (_K * _IDXW), _K, _IDXW).astype(jnp.int32)

    mesh = plsc.VectorSubcoreMesh(core_axis_name="c", subcore_axis_name="s")

    @functools.partial(
        pl.kernel,
        mesh=mesh,
        out_type=jax.ShapeDtypeStruct((seq_len, n_seq, D), jnp.float32),
        scratch_types=[
            pltpu.VMEM((n_chunks, _K, _IDXW), jnp.int32),
            pltpu.VMEM((_NBUF, _CHUNK, D), jnp.float32),
        ] + [pltpu.SemaphoreType.DMA] * (2 * _NBUF),
    )
    def emb(idx_hbm, table_hbm, out_hbm, idx_v, rows_v, *sems):
        gsem = sems[:_NBUF]
        wsem = sems[_NBUF:]
        wid = lax.axis_index("s") * _NC + lax.axis_index("c")
        chunk0 = wid * n_chunks

        pltpu.sync_copy(idx_hbm.at[pl.ds(chunk0, n_chunks)], idx_v)

        def start_chunk(g):
            s = g % _NBUF
            return [
                pltpu.async_copy(
                    table_hbm.at[idx_v.at[g, j]],
                    rows_v.at[s, pl.ds(j * _IDXW, _IDXW), :],
                    gsem[s])
                for j in range(_K)
            ]

        pending_g = {g: start_chunk(g) for g in range(min(_NBUF - 1, n_chunks))}
        pending_wb = [None] * _NBUF
        for g in range(n_chunks):
            s = g % _NBUF
            gn = g + _NBUF - 1
            if gn < n_chunks:
                sn = gn % _NBUF
                if pending_wb[sn] is not None:
                    pending_wb[sn].wait()
                    pending_wb[sn] = None
                pending_g[gn] = start_chunk(gn)
            for d in pending_g.pop(g):
                d.wait()

            def scale_row(i, carry, s=s):
                for j in range(D // _LANES):
                    sl = pl.ds(j * _LANES, _LANES)
                    rows_v[s, i, sl] = rows_v[s, i, sl] * scale
                return carry
            lax.fori_loop(0, _CHUNK, scale_row, 0)

            gchunk = chunk0 + g
            pending_wb[s] = pltpu.async_copy(
                rows_v.at[s],
                out_hbm.at[gchunk // chunks_per_row,
                           pl.ds((gchunk % chunks_per_row) * _CHUNK, _CHUNK),
                           :],
                wsem[s])
        for s in range(_NBUF):
            if pending_wb[s] is not None:
                pending_wb[s].wait()

    out_t = emb(idx3d, lut)
    return jnp.transpose(out_t, (1, 0, 2))
